# baseline (speedup 1.0000x reference)
import jax
import jax.numpy as jnp
from jax.experimental import pallas as pl
from jax.experimental.pallas import tpu as pltpu

_BLOCK_M = 256
_BLOCK_K = 4096


def _make_body(nkc, kb):
    def body(ain_ref, aout_ref, sin_ref, sout_ref, sc_ref, wr_ref, br_ref,
             wz_ref, bz_ref, wh_ref, bh_ref, out_ref, acc_ref):
        j = pl.program_id(1)
        p_in = jnp.dot(ain_ref[...], sin_ref[pl.ds(j * kb, kb), :],
                       preferred_element_type=jnp.float32)
        p_out = jnp.dot(aout_ref[...], sout_ref[pl.ds(j * kb, kb), :],
                        preferred_element_type=jnp.float32)
        p = jnp.concatenate((p_in, p_out), axis=-1)

        @pl.when(j == 0)
        def _():
            acc_ref[...] = p

        @pl.when(j > 0)
        def _():
            acc_ref[...] += p

        @pl.when(j == nkc - 1)
        def _():
            d = sc_ref.shape[1]
            a_in = acc_ref[:, :d]
            a_out = acc_ref[:, d:]
            sc = sc_ref[...]
            acat = jnp.concatenate((a_in, a_out, sc), axis=-1)
            r = jax.nn.sigmoid(jnp.dot(acat, wr_ref[...],
                                       preferred_element_type=jnp.float32)
                               + br_ref[...])
            z = jax.nn.sigmoid(jnp.dot(acat, wz_ref[...],
                                       preferred_element_type=jnp.float32)
                               + bz_ref[...])
            jcat = jnp.concatenate((a_in, a_out, r * sc), axis=-1)
            h_hat = jnp.tanh(jnp.dot(jcat, wh_ref[...],
                                     preferred_element_type=jnp.float32)
                             + bh_ref[...])
            out_ref[...] = (1.0 - z) * sc + z * h_hat

    return body


def kernel(state_in, state_out, state_cur, A, W_r, b_r, W_z, b_z, W_h, b_h):
    s_in = state_in[0]
    s_out = state_out[0]
    n, d = state_cur.shape
    k = s_in.shape[0]
    bm = _BLOCK_M
    kb = _BLOCK_K
    nblk = n // bm
    nkc = k // kb

    A2 = A.reshape(2 * n, k)
    in_specs = [
        pl.BlockSpec((bm, kb), lambda i, j: (i, j)),
        pl.BlockSpec((bm, kb), lambda i, j: (i + nblk, j)),
        pl.BlockSpec((k, d), lambda i, j: (0, 0)),
        pl.BlockSpec((k, d), lambda i, j: (0, 0)),
        pl.BlockSpec((bm, d), lambda i, j: (i, 0)),
        pl.BlockSpec((3 * d, d), lambda i, j: (0, 0)),
        pl.BlockSpec((1, d), lambda i, j: (0, 0)),
        pl.BlockSpec((3 * d, d), lambda i, j: (0, 0)),
        pl.BlockSpec((1, d), lambda i, j: (0, 0)),
        pl.BlockSpec((3 * d, d), lambda i, j: (0, 0)),
        pl.BlockSpec((1, d), lambda i, j: (0, 0)),
    ]
    out = pl.pallas_call(
        _make_body(nkc, kb),
        grid=(nblk, nkc),
        in_specs=in_specs,
        out_specs=pl.BlockSpec((bm, d), lambda i, j: (i, 0)),
        out_shape=jax.ShapeDtypeStruct((n, d), jnp.float32),
        scratch_shapes=[pltpu.VMEM((bm, 2 * d), jnp.float32)],
        compiler_params=pltpu.CompilerParams(
            dimension_semantics=(pltpu.ARBITRARY, pltpu.ARBITRARY)),
    )(A2, A2, s_in, s_out, state_cur,
      W_r, b_r.reshape(1, d), W_z, b_z.reshape(1, d), W_h, b_h.reshape(1, d))
    return out


# speedup vs baseline: 1.0433x; 1.0433x over previous
import jax
import jax.numpy as jnp
from jax.experimental import pallas as pl
from jax.experimental.pallas import tpu as pltpu

_BLOCK_M = 256


def _make_body(nh, bm):
    def body(a_ref, sin_ref, sout_ref, sc_ref, wr_ref, br_ref, wz_ref,
             bz_ref, wh_ref, bh_ref, out_ref, acc_ref):
        i = pl.program_id(0)

        @pl.when(i < nh)
        def _():
            acc_ref[pl.ds(i * bm, bm), :] = jnp.dot(
                a_ref[...], sin_ref[...], preferred_element_type=jnp.float32)

        @pl.when(i >= nh)
        def _():
            a_out = jnp.dot(a_ref[...], sout_ref[...],
                            preferred_element_type=jnp.float32)
            a_in = acc_ref[pl.ds((i - nh) * bm, bm), :]
            sc = sc_ref[...]
            acat = jnp.concatenate((a_in, a_out, sc), axis=-1)
            r = jax.nn.sigmoid(jnp.dot(acat, wr_ref[...],
                                       preferred_element_type=jnp.float32)
                               + br_ref[...])
            z = jax.nn.sigmoid(jnp.dot(acat, wz_ref[...],
                                       preferred_element_type=jnp.float32)
                               + bz_ref[...])
            jcat = jnp.concatenate((a_in, a_out, r * sc), axis=-1)
            h_hat = jnp.tanh(jnp.dot(jcat, wh_ref[...],
                                     preferred_element_type=jnp.float32)
                             + bh_ref[...])
            out_ref[...] = (1.0 - z) * sc + z * h_hat

    return body


def kernel(state_in, state_out, state_cur, A, W_r, b_r, W_z, b_z, W_h, b_h):
    s_in = state_in[0]
    s_out = state_out[0]
    n, d = state_cur.shape
    k = s_in.shape[0]
    bm = _BLOCK_M
    nh = n // bm

    A2 = A.reshape(2 * n, k)
    second = lambda i: (jnp.maximum(i - nh, 0), 0)
    in_specs = [
        pl.BlockSpec((bm, k), lambda i: (i, 0)),
        pl.BlockSpec((k, d), lambda i: (0, 0)),
        pl.BlockSpec((k, d), lambda i: (0, 0)),
        pl.BlockSpec((bm, d), second),
        pl.BlockSpec((3 * d, d), lambda i: (0, 0)),
        pl.BlockSpec((1, d), lambda i: (0, 0)),
        pl.BlockSpec((3 * d, d), lambda i: (0, 0)),
        pl.BlockSpec((1, d), lambda i: (0, 0)),
        pl.BlockSpec((3 * d, d), lambda i: (0, 0)),
        pl.BlockSpec((1, d), lambda i: (0, 0)),
    ]
    out = pl.pallas_call(
        _make_body(nh, bm),
        grid=(2 * nh,),
        in_specs=in_specs,
        out_specs=pl.BlockSpec((bm, d), second),
        out_shape=jax.ShapeDtypeStruct((n, d), jnp.float32),
        scratch_shapes=[pltpu.VMEM((n, d), jnp.float32)],
    )(A2, s_in, s_out, state_cur,
      W_r, b_r.reshape(1, d), W_z, b_z.reshape(1, d), W_h, b_h.reshape(1, d))
    return out


# speedup vs baseline: 1.0494x; 1.0058x over previous
import jax
import jax.numpy as jnp
from jax.experimental import pallas as pl
from jax.experimental.pallas import tpu as pltpu

_BLOCK_M = 512


def _make_body(nh, bm):
    def body(a_ref, sin_ref, sout_ref, sc_ref, wr_ref, br_ref, wz_ref,
             bz_ref, wh_ref, bh_ref, out_ref, acc_ref):
        i = pl.program_id(0)

        @pl.when(i < nh)
        def _():
            acc_ref[pl.ds(i * bm, bm), :] = jnp.dot(
                a_ref[...], sin_ref[...], preferred_element_type=jnp.float32)

        @pl.when(i >= nh)
        def _():
            a_out = jnp.dot(a_ref[...], sout_ref[...],
                            preferred_element_type=jnp.float32)
            a_in = acc_ref[pl.ds((i - nh) * bm, bm), :]
            sc = sc_ref[...]
            acat = jnp.concatenate((a_in, a_out, sc), axis=-1)
            r = jax.nn.sigmoid(jnp.dot(acat, wr_ref[...],
                                       preferred_element_type=jnp.float32)
                               + br_ref[...])
            z = jax.nn.sigmoid(jnp.dot(acat, wz_ref[...],
                                       preferred_element_type=jnp.float32)
                               + bz_ref[...])
            jcat = jnp.concatenate((a_in, a_out, r * sc), axis=-1)
            h_hat = jnp.tanh(jnp.dot(jcat, wh_ref[...],
                                     preferred_element_type=jnp.float32)
                             + bh_ref[...])
            out_ref[...] = (1.0 - z) * sc + z * h_hat

    return body


def kernel(state_in, state_out, state_cur, A, W_r, b_r, W_z, b_z, W_h, b_h):
    s_in = state_in[0]
    s_out = state_out[0]
    n, d = state_cur.shape
    k = s_in.shape[0]
    bm = _BLOCK_M
    nh = n // bm

    A2 = A.reshape(2 * n, k)
    second = lambda i: (jnp.maximum(i - nh, 0), 0)
    in_specs = [
        pl.BlockSpec((bm, k), lambda i: (i, 0)),
        pl.BlockSpec((k, d), lambda i: (0, 0)),
        pl.BlockSpec((k, d), lambda i: (0, 0)),
        pl.BlockSpec((bm, d), second),
        pl.BlockSpec((3 * d, d), lambda i: (0, 0)),
        pl.BlockSpec((1, d), lambda i: (0, 0)),
        pl.BlockSpec((3 * d, d), lambda i: (0, 0)),
        pl.BlockSpec((1, d), lambda i: (0, 0)),
        pl.BlockSpec((3 * d, d), lambda i: (0, 0)),
        pl.BlockSpec((1, d), lambda i: (0, 0)),
    ]
    out = pl.pallas_call(
        _make_body(nh, bm),
        grid=(2 * nh,),
        in_specs=in_specs,
        out_specs=pl.BlockSpec((bm, d), second),
        out_shape=jax.ShapeDtypeStruct((n, d), jnp.float32),
        scratch_shapes=[pltpu.VMEM((n, d), jnp.float32)],
    )(A2, s_in, s_out, state_cur,
      W_r, b_r.reshape(1, d), W_z, b_z.reshape(1, d), W_h, b_h.reshape(1, d))
    return out


# speedup vs baseline: 1.0527x; 1.0031x over previous
import jax
import jax.numpy as jnp
from jax.experimental import pallas as pl

_BLOCK_M = 256


def _body(a_ref, sin_ref, sout_ref, sc_ref, wr_ref, br_ref, wz_ref, bz_ref,
          wh_ref, bh_ref, out_ref):
    a_in = jnp.dot(a_ref[0], sin_ref[...], preferred_element_type=jnp.float32)
    a_out = jnp.dot(a_ref[1], sout_ref[...], preferred_element_type=jnp.float32)
    sc = sc_ref[...]
    acat = jnp.concatenate((a_in, a_out, sc), axis=-1)
    r = jax.nn.sigmoid(jnp.dot(acat, wr_ref[...],
                               preferred_element_type=jnp.float32) + br_ref[...])
    z = jax.nn.sigmoid(jnp.dot(acat, wz_ref[...],
                               preferred_element_type=jnp.float32) + bz_ref[...])
    jcat = jnp.concatenate((a_in, a_out, r * sc), axis=-1)
    h_hat = jnp.tanh(jnp.dot(jcat, wh_ref[...],
                             preferred_element_type=jnp.float32) + bh_ref[...])
    out_ref[...] = (1.0 - z) * sc + z * h_hat


def kernel(state_in, state_out, state_cur, A, W_r, b_r, W_z, b_z, W_h, b_h):
    s_in = state_in[0]
    s_out = state_out[0]
    n, d = state_cur.shape
    k = s_in.shape[0]
    bm = _BLOCK_M

    in_specs = [
        pl.BlockSpec((2, bm, k), lambda i: (0, i, 0)),
        pl.BlockSpec((k, d), lambda i: (0, 0)),
        pl.BlockSpec((k, d), lambda i: (0, 0)),
        pl.BlockSpec((bm, d), lambda i: (i, 0)),
        pl.BlockSpec((3 * d, d), lambda i: (0, 0)),
        pl.BlockSpec((1, d), lambda i: (0, 0)),
        pl.BlockSpec((3 * d, d), lambda i: (0, 0)),
        pl.BlockSpec((1, d), lambda i: (0, 0)),
        pl.BlockSpec((3 * d, d), lambda i: (0, 0)),
        pl.BlockSpec((1, d), lambda i: (0, 0)),
    ]
    out = pl.pallas_call(
        _body,
        grid=(n // bm,),
        in_specs=in_specs,
        out_specs=pl.BlockSpec((bm, d), lambda i: (i, 0)),
        out_shape=jax.ShapeDtypeStruct((n, d), jnp.float32),
    )(A, s_in, s_out, state_cur,
      W_r, b_r.reshape(1, d), W_z, b_z.reshape(1, d), W_h, b_h.reshape(1, d))
    return out
